# baseline (speedup 1.0000x reference)
import functools

import jax
import jax.numpy as jnp
from jax import lax
from jax.experimental import pallas as pl
from jax.experimental.pallas import tpu as pltpu
from jax.experimental.pallas import tpu_sc as plsc

BUCKETS = 1000000
DIM = 64
LANES = 16
NUM_CORES = 2
NUM_SUBCORES = 16
NUM_WORKERS = NUM_CORES * NUM_SUBCORES
HALO = 8


def _sc_embed(token_flat, scale_vec, embed_weight, *, bt, chunk):
    n_grp = chunk // LANES
    n_batch = chunk // 128
    chunks_per_row = 8192 // chunk
    mod = BUCKETS - 1

    mesh = plsc.VectorSubcoreMesh(core_axis_name="c", subcore_axis_name="s")

    @functools.partial(
        pl.kernel,
        out_type=jax.ShapeDtypeStruct((bt, DIM), jnp.float32),
        mesh=mesh,
        scratch_types=[
            pltpu.VMEM((HALO + chunk,), jnp.int32),
            pltpu.VMEM((n_batch, 128), jnp.int32),
            pltpu.VMEM((chunk, DIM), jnp.float32),
            pltpu.VMEM((LANES,), jnp.float32),
            pltpu.SemaphoreType.DMA,
        ],
        compiler_params=pltpu.CompilerParams(use_tc_tiling_on_sc=False),
    )
    def body(tok_hbm, scale_hbm, table_hbm, out_hbm,
             tok_v, idx_v, rows_v, scale_v, sem):
        wid = lax.axis_index("s") * NUM_CORES + lax.axis_index("c")
        base = wid * chunk
        at_row_start = (wid % chunks_per_row) == 0

        pltpu.sync_copy(scale_hbm, scale_v)

        @pl.when(at_row_start)
        def _():
            pltpu.sync_copy(tok_hbm.at[pl.ds(base, chunk)],
                            tok_v.at[pl.ds(HALO, chunk)])

        @pl.when(jnp.logical_not(at_row_start))
        def _():
            pltpu.sync_copy(tok_hbm.at[pl.ds(base - HALO, HALO + chunk)],
                            tok_v)

        lanes = lax.iota(jnp.int32, LANES)
        pos_in_row = (wid % chunks_per_row) * chunk + lanes

        def hash_group(i, _):
            q = i * LANES
            t2 = tok_v[pl.ds(q + HALO, LANES)]
            t1 = tok_v[pl.ds(q + HALO - 1, LANES)]
            t0 = tok_v[pl.ds(q + HALO - 2, LANES)]
            h = 131071 * t2 + 524287 * t1 + 8191 * t0
            m = h % mod
            m = jnp.where(pos_in_row + q < 2, mod, m)
            idx_v[i // 8, pl.ds((i % 8) * LANES, LANES)] = m
            return 0

        lax.fori_loop(0, n_grp, hash_group, 0, unroll=8)

        copies = [
            pltpu.async_copy(table_hbm.at[idx_v.at[j]],
                             rows_v.at[pl.ds(j * 128, 128)], sem)
            for j in range(n_batch)
        ]
        for c in copies:
            c.wait()

        sv = scale_v[...]

        def scale_row(r, _):
            for k in range(DIM // LANES):
                rows_v[r, pl.ds(k * LANES, LANES)] = (
                    rows_v[r, pl.ds(k * LANES, LANES)] * sv)
            return 0

        lax.fori_loop(0, chunk, scale_row, 0, unroll=4)

        pltpu.sync_copy(rows_v, out_hbm.at[pl.ds(base, chunk)])

    return body(token_flat, scale_vec, embed_weight)


def kernel(token_ids, embed_weight, scale):
    b, t = token_ids.shape
    bt = b * t
    tok_flat = token_ids.reshape(bt).astype(jnp.int32)
    scale_vec = jnp.full((LANES,), scale, dtype=jnp.float32)
    out = _sc_embed(tok_flat, scale_vec, embed_weight,
                    bt=bt, chunk=bt // NUM_WORKERS)
    return out.reshape(b, t, DIM)


# speedup vs baseline: 2.1497x; 2.1497x over previous
import functools

import jax
import jax.numpy as jnp
from jax import lax
from jax.experimental import pallas as pl
from jax.experimental.pallas import tpu as pltpu
from jax.experimental.pallas import tpu_sc as plsc

BUCKETS = 1000000
DIM = 64
LANES = 16
NUM_CORES = 2
NUM_SUBCORES = 16
NUM_WORKERS = NUM_CORES * NUM_SUBCORES
HALO = 8
W = 32


def _sc_embed(token_ids, scale_vec, table3d, *, b, t):
    bt = b * t
    chunk = bt // NUM_WORKERS
    n_grp = chunk // LANES
    n_win = chunk // W
    chunks_per_row = t // chunk
    mod = BUCKETS - 1

    mesh = plsc.VectorSubcoreMesh(core_axis_name="c", subcore_axis_name="s")

    @functools.partial(
        pl.kernel,
        out_type=jax.ShapeDtypeStruct((bt, DIM), jnp.float32),
        mesh=mesh,
        scratch_types=[
            pltpu.VMEM((HALO + chunk,), jnp.int32),
            pltpu.VMEM((n_win, W), jnp.int32),
            pltpu.VMEM((n_win, W), jnp.int32),
            pltpu.VMEM((2, W, 8, DIM), jnp.float32),
            pltpu.VMEM((2, W, DIM), jnp.float32),
            pltpu.VMEM((LANES,), jnp.float32),
            pltpu.SemaphoreType.DMA,
            pltpu.SemaphoreType.DMA,
        ],
    )
    def body(tok_hbm, scale_hbm, table_hbm, out_hbm,
             tok_v, tidx_v, sidx_v, tiles_v, orows_v, scale_v,
             gsem, osem):
        wid = lax.axis_index("s") * NUM_CORES + lax.axis_index("c")
        cpos = (wid % chunks_per_row) * chunk
        base = wid * chunk
        at_row_start = cpos == 0

        pltpu.sync_copy(scale_hbm, scale_v)

        @pl.when(at_row_start)
        def _():
            pltpu.sync_copy(tok_hbm.at[pl.ds(base, chunk)],
                            tok_v.at[pl.ds(HALO, chunk)])

        @pl.when(jnp.logical_not(at_row_start))
        def _():
            pltpu.sync_copy(tok_hbm.at[pl.ds(base - HALO, HALO + chunk)],
                            tok_v)

        lanes = lax.iota(jnp.int32, LANES)
        pos_in_row = cpos + lanes

        def hash_group(i, _):
            q = i * LANES
            t2 = tok_v[pl.ds(q + HALO, LANES)]
            t1 = tok_v[pl.ds(q + HALO - 1, LANES)]
            t0 = tok_v[pl.ds(q + HALO - 2, LANES)]
            h = 131071 * t2 + 524287 * t1 + 8191 * t0
            m = h % mod
            m = jnp.where(pos_in_row + q < 2, mod, m)
            j = i // (W // LANES)
            col = (i % (W // LANES)) * LANES
            tidx_v[j, pl.ds(col, LANES)] = m >> 3
            sidx_v[j, pl.ds(col, LANES)] = m & 7
            return 0

        lax.fori_loop(0, n_grp, hash_group, 0, unroll=8)

        sv = scale_v[...]

        def fire(j, buf):
            for g in range(W // LANES):
                tvec = tidx_v[j, pl.ds(g * LANES, LANES)]
                for l in range(LANES):
                    k = g * LANES + l
                    pltpu.async_copy(table_hbm.at[pl.ds(tvec[l], 1)],
                                     tiles_v.at[buf, pl.ds(k, 1)], gsem)

        def extract(j, buf):
            for g in range(W // LANES):
                svec = sidx_v[j, pl.ds(g * LANES, LANES)]
                for l in range(LANES):
                    k = g * LANES + l
                    s = svec[l]
                    for c in range(DIM // LANES):
                        orows_v[buf, k, pl.ds(c * LANES, LANES)] = (
                            tiles_v[buf, k, s, pl.ds(c * LANES, LANES)] * sv)
            return pltpu.async_copy(
                orows_v.at[buf],
                out_hbm.at[pl.ds(base + j * W, W)], osem)

        def wait_gather(buf):
            pltpu.make_async_copy(table_hbm.at[pl.ds(0, W)],
                                  tiles_v.at[buf], gsem).wait()

        fire(0, 0)

        def step(jj, _):
            j0 = jj * 2
            fire(j0 + 1, 1)
            wait_gather(0)
            o0 = extract(j0, 0)

            @pl.when(j0 + 2 < n_win)
            def _():
                fire(j0 + 2, 0)

            wait_gather(1)
            o1 = extract(j0 + 1, 1)
            o0.wait()
            o1.wait()
            return 0

        lax.fori_loop(0, n_win // 2, step, 0)

    return body(token_ids, scale_vec, table3d)


def kernel(token_ids, embed_weight, scale):
    b, t = token_ids.shape
    table3d = embed_weight.reshape(BUCKETS // 8, 8, DIM)
    scale_vec = jnp.full((LANES,), scale, dtype=jnp.float32)
    tok_flat = token_ids.reshape(b * t).astype(jnp.int32)
    out = _sc_embed(tok_flat, scale_vec, table3d, b=b, t=t)
    return out.reshape(b, t, DIM)
